# baseline (speedup 1.0000x reference)
import functools

import jax
import jax.numpy as jnp
from jax.experimental import pallas as pl
from jax.experimental.pallas import tpu as pltpu

_BIG = 1e9

_DN_T = (((0,), (0,)), ((), ()))


def _hawkes_body(N, B, SUB, K, M,
                 tmi_ref, alpha_ref, mu_ref, gammav_ref, gammac_ref,
                 gamma_ref, tf_ref, anch_ref, panch_ref,
                 out_ref, carry_ref):
    c = pl.program_id(0)
    Tf = tf_ref[0, 0]

    @pl.when(c == 0)
    def _init():
        out_ref[0, 0] = Tf * jnp.sum(mu_ref[...])
        carry_ref[...] = jnp.zeros_like(carry_ref)

    ii = jax.lax.broadcasted_iota(jnp.int32, (B, B), 0)
    jj = jax.lax.broadcasted_iota(jnp.int32, (B, B), 1)
    tri = ii > jj
    miota = jax.lax.broadcasted_iota(jnp.int32, (B, M), 1).astype(jnp.float32)
    grow = gammav_ref[...]
    gcol = gammac_ref[...]
    ones1B = jnp.ones((1, B), jnp.float32)
    ones1M = jnp.ones((1, M), jnp.float32)

    Cval = carry_ref[...]
    contrib = jnp.zeros((B, 1), jnp.float32)
    for s in range(SUB):
        tr = tmi_ref[0, 0:1, s * B:(s + 1) * B]
        mir = tmi_ref[0, 1:2, s * B:(s + 1) * B]
        b_prev = panch_ref[0, c * SUB + s]
        b_new = anch_ref[0, c * SUB + s]

        A = jnp.concatenate([tr, ones1B], axis=0)
        B2 = jnp.concatenate([ones1B, -tr], axis=0)

        mic_b = jax.lax.dot_general(mir, ones1M, _DN_T,
                                    preferred_element_type=jnp.float32)
        P = (mic_b == miota).astype(jnp.float32)
        P_bf = P.astype(jnp.bfloat16)

        Vrows = [jnp.dot(Cval[k:k + 1, :].astype(jnp.bfloat16), alpha_ref[k],
                         preferred_element_type=jnp.float32)
                 for k in range(K)]
        V = jnp.concatenate(Vrows, axis=0)
        Wd = jnp.concatenate([-grow, b_prev * grow], axis=0)
        dcross = jnp.exp(jax.lax.dot_general(
            A, Wd, _DN_T, preferred_element_type=jnp.float32))
        Yacc = jnp.dot(dcross, V, preferred_element_type=jnp.float32)

        gW = jnp.concatenate([gcol, -b_new * gcol], axis=1)
        F = jnp.exp(jnp.dot(gW, A, preferred_element_type=jnp.float32))
        G = jax.lax.dot_general(F, P, (((1,), (0,)), ((), ())),
                                preferred_element_type=jnp.float32)
        dblk = jnp.exp(-(b_new - b_prev) * gcol)
        Cval = dblk * Cval + G

        delta_raw = jax.lax.dot_general(A, B2, _DN_T,
                                        preferred_element_type=jnp.float32)
        delta = jnp.where(tri, delta_raw, _BIG)
        for k in range(K):
            gk = gamma_ref[0, k]
            Ek = jnp.exp(-gk * delta).astype(jnp.bfloat16)
            PAk = jnp.dot(P_bf, alpha_ref[k],
                          preferred_element_type=jnp.float32
                          ).astype(jnp.bfloat16)
            Yacc += jnp.dot(Ek, PAk, preferred_element_type=jnp.float32)

        lam = jnp.sum((Yacc + mu_ref[...]) * P, axis=1, keepdims=True)
        gidx = (jax.lax.broadcasted_iota(jnp.int32, (B, 1), 0)
                + (c * SUB + s) * B)
        lam_safe = jnp.where(gidx < N, lam, 1.0)

        As_g = jnp.sum(alpha_ref[...].astype(jnp.float32), axis=2)
        Wc = jnp.concatenate([grow, -Tf * grow], axis=0)
        CKp = (1.0 - jnp.exp(jax.lax.dot_general(
            A, Wc, _DN_T, preferred_element_type=jnp.float32))) / grow
        PA = jax.lax.dot_general(P, As_g, (((1,), (1,)), ((), ())),
                                 preferred_element_type=jnp.float32)
        contrib += (jnp.sum(PA * CKp, axis=1, keepdims=True)
                    - jnp.log(lam_safe))

    carry_ref[...] = Cval
    out_ref[0, 0] += jnp.sum(contrib)


def kernel(mu, alpha, gamma, ti, mi, T):
    N = ti.shape[1]
    M = mu.shape[0]
    K = gamma.shape[0]
    B = 256
    SUB = 8
    BS = B * SUB
    C = -(-N // BS)
    NP = C * BS
    pad = NP - N
    CS = C * SUB

    t = ti.reshape(N).astype(jnp.float32)
    micf = mi.astype(jnp.float32)
    if pad:
        t_pad = jnp.concatenate([t, jnp.broadcast_to(t[N - 1], (pad,))])
        micf = jnp.concatenate([micf, jnp.full((pad,), -1.0, jnp.float32)])
    else:
        t_pad = t

    tmi = jnp.stack([t_pad.reshape(C, BS), micf.reshape(C, BS)], axis=1)
    anchors = t_pad[B - 1::B].reshape(1, CS)
    prev_anchors = jnp.concatenate(
        [jnp.zeros((1, 1), jnp.float32), anchors[:, :-1]], axis=1)
    gamma_f = gamma.astype(jnp.float32)
    gamma_row = gamma_f.reshape(1, K)
    mu2 = mu.reshape(1, M).astype(jnp.float32)
    alpha_g = (alpha.astype(jnp.float32)
               * gamma_f[:, None, None]).astype(jnp.bfloat16)
    Tf = jnp.asarray(T, jnp.float32).reshape(1, 1)

    body = functools.partial(_hawkes_body, N, B, SUB, K, M)
    out = pl.pallas_call(
        body,
        grid=(C,),
        in_specs=[
            pl.BlockSpec((1, 2, BS), lambda c: (c, 0, 0)),
            pl.BlockSpec((K, M, M), lambda c: (0, 0, 0)),
            pl.BlockSpec((1, M), lambda c: (0, 0)),
            pl.BlockSpec((1, K), lambda c: (0, 0)),
            pl.BlockSpec((K, 1), lambda c: (0, 0)),
            pl.BlockSpec(memory_space=pltpu.SMEM),
            pl.BlockSpec(memory_space=pltpu.SMEM),
            pl.BlockSpec(memory_space=pltpu.SMEM),
            pl.BlockSpec(memory_space=pltpu.SMEM),
        ],
        out_specs=pl.BlockSpec(memory_space=pltpu.SMEM),
        out_shape=jax.ShapeDtypeStruct((1, 1), jnp.float32),
        scratch_shapes=[pltpu.VMEM((K, M), jnp.float32)],
    )(tmi, alpha_g, mu2, gamma_row, gamma_row.reshape(K, 1),
      gamma_row, Tf, anchors, prev_anchors)
    return out[0, 0] / jnp.float32(N)
